# baseline (speedup 1.0000x reference)
import functools

import jax
import jax.numpy as jnp
from jax import lax
from jax.experimental import pallas as pl
from jax.experimental.pallas import tpu as pltpu

E = 64
K = 8
H = 768


def _router_body(x_ref, w_ref, b_ref, idx_ref, wgt_ref):
    logits = lax.dot_general(
        w_ref[...], x_ref[...],
        dimension_numbers=(((1,), (1,)), ((), ())),
        preferred_element_type=jnp.float32,
    )
    scores = jax.nn.sigmoid(logits)
    sel_f = scores + b_ref[...]

    T = scores.shape[1]
    eid = lax.broadcasted_iota(jnp.int32, (E, T), 0)
    sel = sel_f
    neg_inf = jnp.float32(-jnp.inf)

    picked_scores = []
    picked_idx = []
    for k in range(K):
        m = jnp.max(sel, axis=0, keepdims=True)
        is_max = sel == m
        idx = jnp.min(jnp.where(is_max, eid, E), axis=0, keepdims=True)
        hit = eid == idx
        score_k = jnp.sum(jnp.where(hit, scores, 0.0), axis=0, keepdims=True)
        picked_scores.append(score_k)
        picked_idx.append(idx)
        sel = jnp.where(hit, neg_inf, sel)

    stacked = jnp.concatenate(picked_scores, axis=0)
    total = jnp.sum(stacked, axis=0, keepdims=True)
    idx_ref[...] = jnp.concatenate(picked_idx, axis=0).T
    wgt_ref[...] = (stacked / total).T


@functools.partial(jax.jit, static_argnames=("block_t",))
def _router(x2d, W, bias, block_t=2048):
    n_tok = x2d.shape[0]
    grid = (n_tok // block_t,)
    idx_t, wgt_t = pl.pallas_call(
        _router_body,
        grid=grid,
        in_specs=[
            pl.BlockSpec((block_t, H), lambda i: (i, 0)),
            pl.BlockSpec((E, H), lambda i: (0, 0)),
            pl.BlockSpec((E, 1), lambda i: (0, 0)),
        ],
        out_specs=[
            pl.BlockSpec((block_t, K), lambda i: (i, 0)),
            pl.BlockSpec((block_t, K), lambda i: (i, 0)),
        ],
        out_shape=[
            jax.ShapeDtypeStruct((n_tok, K), jnp.int32),
            jax.ShapeDtypeStruct((n_tok, K), jnp.float32),
        ],
        compiler_params=pltpu.CompilerParams(
            dimension_semantics=("parallel",),
        ),
    )(x2d, W, bias)
    return idx_t, wgt_t


def kernel(x, W, expert_bias):
    B, S, _ = x.shape
    x2d = x.reshape(B * S, H)
    idx_t, wgt_t = _router(x2d, W, expert_bias.reshape(E, 1))
    top_k_indices = idx_t.reshape(B, S, K)
    top_k_weights = wgt_t.reshape(B, S, K)
    return (top_k_indices, top_k_weights)


# speedup vs baseline: 1.6183x; 1.6183x over previous
import functools

import jax
import jax.numpy as jnp
from jax import lax
from jax.experimental import pallas as pl
from jax.experimental.pallas import tpu as pltpu

E = 64
K = 8
H = 768


def _router_body(x_ref, w_ref, b_ref, idx_ref, wgt_ref):
    logits = lax.dot_general(
        w_ref[...], x_ref[...],
        dimension_numbers=(((1,), (1,)), ((), ())),
        preferred_element_type=jnp.float32,
    )
    scores = jax.nn.sigmoid(logits)
    sel_f = scores + b_ref[...]

    T = scores.shape[1]
    eid = lax.broadcasted_iota(jnp.int32, (E, T), 0)
    sel = sel_f
    neg_inf = jnp.float32(-jnp.inf)

    picked_scores = []
    for k in range(K):
        m = jnp.max(sel, axis=0, keepdims=True)
        is_max = sel == m
        idx = jnp.min(jnp.where(is_max, eid, E), axis=0, keepdims=True)
        hit = eid == idx
        score_k = jnp.sum(jnp.where(hit, scores, 0.0), axis=0, keepdims=True)
        picked_scores.append(score_k)
        idx_ref[k : k + 1, :] = idx
        sel = jnp.where(hit, neg_inf, sel)

    stacked = jnp.concatenate(picked_scores, axis=0)
    total = jnp.sum(stacked, axis=0, keepdims=True)
    wgt_ref[...] = stacked / total


@functools.partial(jax.jit, static_argnames=("block_t",))
def _router(x2d, W, bias, block_t=4096):
    n_tok = x2d.shape[0]
    grid = (n_tok // block_t,)
    idx_t, wgt_t = pl.pallas_call(
        _router_body,
        grid=grid,
        in_specs=[
            pl.BlockSpec((block_t, H), lambda i: (i, 0)),
            pl.BlockSpec((E, H), lambda i: (0, 0)),
            pl.BlockSpec((E, 1), lambda i: (0, 0)),
        ],
        out_specs=[
            pl.BlockSpec((K, block_t), lambda i: (0, i)),
            pl.BlockSpec((K, block_t), lambda i: (0, i)),
        ],
        out_shape=[
            jax.ShapeDtypeStruct((K, n_tok), jnp.int32),
            jax.ShapeDtypeStruct((K, n_tok), jnp.float32),
        ],
        compiler_params=pltpu.CompilerParams(
            dimension_semantics=("parallel",),
        ),
    )(x2d, W, bias)
    return idx_t, wgt_t


def kernel(x, W, expert_bias):
    B, S, _ = x.shape
    x2d = x.reshape(B * S, H)
    idx_t, wgt_t = _router(x2d, W, expert_bias.reshape(E, 1))
    top_k_indices = idx_t.T.reshape(B, S, K)
    top_k_weights = wgt_t.T.reshape(B, S, K)
    return (top_k_indices, top_k_weights)
